# baseline (speedup 1.0000x reference)
import functools

import jax
import jax.numpy as jnp
from jax import lax
from jax.experimental import pallas as pl
from jax.experimental.pallas import tpu as pltpu
from jax.experimental.pallas import tpu_sc as plsc

NUM_HEADS = 32
NPOS = 64 * 64
LANES = 16
NGROUPS = NPOS // 64


def _sc_gather(table, rank_p, file_p):
    mesh = plsc.VectorSubcoreMesh(core_axis_name="c", subcore_axis_name="s")

    @functools.partial(
        pl.kernel,
        mesh=mesh,
        out_type=jax.ShapeDtypeStruct((NUM_HEADS, NPOS), jnp.float32),
        scratch_types=[
            pltpu.VMEM((15, 15), jnp.float32),
            pltpu.VMEM((NPOS // 4,), jnp.int32),
            pltpu.VMEM((NPOS // 4,), jnp.int32),
            pltpu.VMEM((NPOS,), jnp.float32),
            pltpu.SemaphoreType.DMA,
            pltpu.SemaphoreType.DMA,
            pltpu.SemaphoreType.DMA,
        ],
        compiler_params=pltpu.CompilerParams(needs_layout_passes=False),
    )
    def run(table_hbm, rank_hbm, file_hbm, out_hbm,
            table_v, rank_v, file_v, out_v, sem_t, sem_r, sem_f):
        wid = lax.axis_index("s") * 2 + lax.axis_index("c")
        ct = pltpu.async_copy(table_hbm.at[wid], table_v, sem_t)
        cr = pltpu.async_copy(rank_hbm, rank_v, sem_r)
        cf = pltpu.async_copy(file_hbm, file_v, sem_f)
        ct.wait()
        cr.wait()
        cf.wait()

        lane4 = lax.iota(jnp.int32, LANES) * 4

        def body(g, carry):
            rw = rank_v[pl.ds(g * LANES, LANES)]
            fw = file_v[pl.ds(g * LANES, LANES)]
            pos = g * 64 + lane4
            for j in range(4):
                rb = lax.shift_right_logical(rw, 8 * j) & 0xFF
                fb = lax.shift_right_logical(fw, 8 * j) & 0xFF
                plsc.store_scatter(out_v, [pos + j],
                                   plsc.load_gather(table_v, [rb, fb]))
            return carry

        lax.fori_loop(0, NGROUPS, body, 0)
        pltpu.sync_copy(out_v, out_hbm.at[wid])

    return run(table, rank_p, file_p)


def _pack_u8(idx):
    v = idx.reshape(NPOS // 4, 4).astype(jnp.uint8)
    return lax.bitcast_convert_type(v, jnp.int32)


def kernel(bias_table, rank_idx, file_idx):
    out = _sc_gather(bias_table, _pack_u8(rank_idx), _pack_u8(file_idx))
    return out.reshape(NUM_HEADS, 64, 64)
